# baseline (speedup 1.0000x reference)
import functools

import jax
import jax.numpy as jnp
from jax import lax
from jax.experimental import pallas as pl
from jax.experimental.pallas import tpu as pltpu
from jax.experimental.pallas import tpu_sc as plsc

NUM_CLASSES = 1000000
D = 16
B = 16384

_info = plsc.get_sparse_core_info()
_NC = _info.num_cores
_NS = _info.num_subcores
_L = _info.num_lanes
NW = _NC * _NS
BPW = B // NW
_H = 8
NHG = BPW // _H
NITER = (NHG - 1) // 3


def _sc_body(idx_hbm, src_hbm, ow_hbm, epsT_hbm, meanT_hbm, logstdT_hbm,
             outT_hbm, idx_v, src_v, eps_v, ow_v, out_v, *banks_and_sems):
    banks = []
    for k in range(3):
        banks.append((banks_and_sems[2 * k], banks_and_sems[2 * k + 1],
                      banks_and_sems[6 + k]))
    wid = lax.axis_index("s") * _NC + lax.axis_index("c")
    col0 = wid * BPW
    pltpu.sync_copy(idx_hbm.at[wid], idx_v.at[pl.ds(0, BPW)])
    pltpu.sync_copy(src_hbm.at[wid], src_v.at[pl.ds(0, BPW)])
    pltpu.sync_copy(epsT_hbm.at[:, pl.ds(col0, BPW)], eps_v)
    pltpu.sync_copy(ow_hbm.at[pl.ds(col0, BPW)], ow_v.at[pl.ds(0, BPW)])
    iota = lax.iota(jnp.int32, _L)

    def fire(h, bank):
        mbank, lbank, sem = bank
        idxvec = idx_v[pl.ds(h * _H, _L)]
        srcvec = src_v[pl.ds(h * _H, _L)]
        for j in range(_H):
            @pl.when(srcvec[j] == j)
            def _():
                c = pl.multiple_of((idxvec[j] >> 7) << 7, 128)
                pltpu.async_copy(meanT_hbm.at[:, pl.ds(c, 128)],
                                 mbank.at[:, pl.ds(j * 128, 128)], sem)
                pltpu.async_copy(logstdT_hbm.at[:, pl.ds(c, 128)],
                                 lbank.at[:, pl.ds(j * 128, 128)], sem)

    def consume(h, bank):
        mbank, lbank, sem = bank
        idxvec = idx_v[pl.ds(h * _H, _L)]
        srcvec = src_v[pl.ds(h * _H, _L)]
        lanes = idxvec & 127
        owvec = ow_v[pl.ds(h * _H, _L)]
        for j in range(_H):
            @pl.when(srcvec[j] == j)
            def _():
                pltpu.make_async_copy(meanT_hbm.at[:, pl.ds(0, 128)],
                                      mbank.at[:, pl.ds(j * 128, 128)],
                                      sem).wait()
                pltpu.make_async_copy(logstdT_hbm.at[:, pl.ds(0, 128)],
                                      lbank.at[:, pl.ds(j * 128, 128)],
                                      sem).wait()
        for j in range(_H):
            bvec = jnp.broadcast_to(h * _H + j, (_L,))
            pos = jnp.broadcast_to(srcvec[j] * 128 + lanes[j], (_L,))
            w = jnp.broadcast_to(owvec[j], (_L,))
            mcol = plsc.load_gather(mbank, [iota, pos])
            lcol = plsc.load_gather(lbank, [iota, pos])
            ecol = plsc.load_gather(eps_v, [iota, bvec])
            res = (mcol + jnp.exp(lcol) * ecol) * w
            plsc.store_scatter(out_v, [iota, bvec], res)

    fire(0, banks[0])
    fire(1, banks[1])

    def body(t, carry):
        h = 3 * t
        fire(h + 2, banks[2])
        consume(h, banks[0])
        fire(h + 3, banks[0])
        consume(h + 1, banks[1])
        fire(jnp.minimum(h + 4, NHG - 1), banks[1])
        consume(h + 2, banks[2])
        return carry

    lax.fori_loop(0, NITER, body, 0)
    consume(NHG - 1, banks[0])
    mbank1, lbank1, sem1 = banks[1]
    srcvec1 = src_v[pl.ds((NHG - 1) * _H, _L)]
    for j in range(_H):
        @pl.when(srcvec1[j] == j)
        def _():
            pltpu.make_async_copy(meanT_hbm.at[:, pl.ds(0, 128)],
                                  mbank1.at[:, pl.ds(j * 128, 128)],
                                  sem1).wait()
            pltpu.make_async_copy(meanT_hbm.at[:, pl.ds(0, 128)],
                                  lbank1.at[:, pl.ds(j * 128, 128)],
                                  sem1).wait()
    pltpu.sync_copy(out_v, outT_hbm.at[:, pl.ds(col0, BPW)])


def kernel(indices, session_obs_w, eps, variational_mean, variational_logstd):
    order = jnp.argsort(indices.reshape(B))
    sidx = indices.reshape(B)[order]
    ow_s = session_obs_w.reshape(B)[order]
    eps_s = eps.reshape(B, D)[order]
    w8 = (sidx >> 7).reshape(B // _H, _H)
    src = jnp.argmax(w8[:, :, None] == w8[:, None, :], axis=1)
    src2 = src.astype(jnp.int32).reshape(NW, BPW)
    idx2 = sidx.reshape(NW, BPW)
    mesh = plsc.VectorSubcoreMesh(core_axis_name="c", subcore_axis_name="s")
    scratch = [
        pltpu.VMEM((BPW + _L,), jnp.int32),
        pltpu.VMEM((BPW + _L,), jnp.int32),
        pltpu.VMEM((D, BPW), jnp.float32),
        pltpu.VMEM((BPW + _L,), jnp.float32),
        pltpu.VMEM((D, BPW), jnp.float32),
    ]
    scratch += [pltpu.VMEM((D, _H * 128), jnp.float32) for _ in range(6)]
    scratch += [pltpu.SemaphoreType.DMA for _ in range(3)]
    f = functools.partial(
        pl.kernel,
        out_type=jax.ShapeDtypeStruct((D, B), jnp.float32),
        mesh=mesh,
        scratch_types=scratch,
        compiler_params=pltpu.CompilerParams(
            use_tc_tiling_on_sc=True, needs_layout_passes=False),
    )(_sc_body)
    outT = f(idx2, src2, ow_s, eps_s.T, variational_mean.T,
             variational_logstd.T)
    return jnp.zeros((B, D), jnp.float32).at[order].set(outT.T)


# speedup vs baseline: 1.3328x; 1.3328x over previous
import functools

import jax
import jax.numpy as jnp
from jax import lax
from jax.experimental import pallas as pl
from jax.experimental.pallas import tpu as pltpu
from jax.experimental.pallas import tpu_sc as plsc

NUM_CLASSES = 1000000
D = 16
B = 16384

_info = plsc.get_sparse_core_info()
_NC = _info.num_cores
_NS = _info.num_subcores
_L = _info.num_lanes
NW = _NC * _NS
BPW = B // NW
_H = 8
NHG = BPW // _H
NITER = (NHG - 1) // 3


def _sc_body(idx_hbm, ow_hbm, epsT_hbm, meanT_hbm, logstdT_hbm, outT_hbm,
             idx_v, eps_v, ow_v, out_v, *slots_and_sems):
    banks = []
    for k in range(3):
        off = k * 2 * _H
        banks.append((slots_and_sems[off:off + _H],
                      slots_and_sems[off + _H:off + 2 * _H],
                      slots_and_sems[6 * _H + k]))
    wid = lax.axis_index("s") * _NC + lax.axis_index("c")
    col0 = wid * BPW
    pltpu.sync_copy(idx_hbm.at[wid], idx_v.at[pl.ds(0, BPW)])
    pltpu.sync_copy(epsT_hbm.at[:, pl.ds(col0, BPW)], eps_v)
    pltpu.sync_copy(ow_hbm.at[pl.ds(col0, BPW)], ow_v.at[pl.ds(0, BPW)])
    iota = lax.iota(jnp.int32, _L)

    def fire(h, bank):
        mslots, lslots, sem = bank
        idxvec = idx_v[pl.ds(h * _H, _L)]
        for j in range(_H):
            c = pl.multiple_of((idxvec[j] >> 7) << 7, 128)
            pltpu.async_copy(meanT_hbm.at[:, pl.ds(c, 128)], mslots[j], sem)
            pltpu.async_copy(logstdT_hbm.at[:, pl.ds(c, 128)], lslots[j], sem)

    def consume(h, bank):
        mslots, lslots, sem = bank
        idxvec = idx_v[pl.ds(h * _H, _L)]
        lanes = idxvec & 127
        owvec = ow_v[pl.ds(h * _H, _L)]
        for j in range(_H):
            pltpu.make_async_copy(meanT_hbm.at[:, pl.ds(0, 128)],
                                  mslots[j], sem).wait()
            pltpu.make_async_copy(logstdT_hbm.at[:, pl.ds(0, 128)],
                                  lslots[j], sem).wait()
        for j in range(_H):
            bvec = jnp.broadcast_to(h * _H + j, (_L,))
            lane = jnp.broadcast_to(lanes[j], (_L,))
            w = jnp.broadcast_to(owvec[j], (_L,))
            mcol = plsc.load_gather(mslots[j], [iota, lane])
            lcol = plsc.load_gather(lslots[j], [iota, lane])
            ecol = plsc.load_gather(eps_v, [iota, bvec])
            res = (mcol + jnp.exp(lcol) * ecol) * w
            plsc.store_scatter(out_v, [iota, bvec], res)

    fire(0, banks[0])
    fire(1, banks[1])

    def body(t, carry):
        h = 3 * t
        fire(h + 2, banks[2])
        consume(h, banks[0])
        fire(h + 3, banks[0])
        consume(h + 1, banks[1])
        fire(jnp.minimum(h + 4, NHG - 1), banks[1])
        consume(h + 2, banks[2])
        return carry

    lax.fori_loop(0, NITER, body, 0)
    consume(NHG - 1, banks[0])
    mslots1, lslots1, sem1 = banks[1]
    for j in range(_H):
        pltpu.make_async_copy(meanT_hbm.at[:, pl.ds(0, 128)],
                              mslots1[j], sem1).wait()
        pltpu.make_async_copy(meanT_hbm.at[:, pl.ds(0, 128)],
                              lslots1[j], sem1).wait()
    pltpu.sync_copy(out_v, outT_hbm.at[:, pl.ds(col0, BPW)])


def kernel(indices, session_obs_w, eps, variational_mean, variational_logstd):
    idx2 = indices.reshape(NW, BPW)
    ow = session_obs_w.reshape(B)
    mesh = plsc.VectorSubcoreMesh(core_axis_name="c", subcore_axis_name="s")
    scratch = [
        pltpu.VMEM((BPW + _L,), jnp.int32),
        pltpu.VMEM((D, BPW), jnp.float32),
        pltpu.VMEM((BPW + _L,), jnp.float32),
        pltpu.VMEM((D, BPW), jnp.float32),
    ]
    scratch += [pltpu.VMEM((D, 128), jnp.float32) for _ in range(6 * _H)]
    scratch += [pltpu.SemaphoreType.DMA for _ in range(3)]
    f = functools.partial(
        pl.kernel,
        out_type=jax.ShapeDtypeStruct((D, B), jnp.float32),
        mesh=mesh,
        scratch_types=scratch,
        compiler_params=pltpu.CompilerParams(
            use_tc_tiling_on_sc=True, needs_layout_passes=False),
    )(_sc_body)
    outT = f(idx2, ow, eps.T, variational_mean.T, variational_logstd.T)
    return outT.T
